# baseline (speedup 1.0000x reference)
import functools

import jax
import jax.numpy as jnp
from jax import lax
from jax.experimental import pallas as pl
from jax.experimental.pallas import tpu as pltpu
from jax.experimental.pallas import tpu_sc as plsc

_B, _V, _C, _H, _W = 16, 8, 3, 256, 256
_NIN, _NTG = 4, 6
_ROW = _C * _H * _W
_NC, _NS = 2, 16
_NWORK = _NC * _NS
_IN_PW = _B * _NIN // _NWORK
_TG_PW = _B * _NTG // _NWORK


_PERM = [[6, 4, 0, 3, 5, 2], [2, 6, 0, 7, 1, 5], [6, 2, 5, 4, 1, 7],
         [2, 3, 0, 7, 4, 6], [7, 6, 0, 3, 2, 5], [5, 0, 2, 3, 1, 4],
         [1, 4, 3, 7, 2, 6], [4, 5, 1, 7, 0, 2], [1, 3, 7, 5, 6, 0],
         [2, 5, 1, 6, 4, 0], [6, 4, 5, 0, 3, 2], [3, 4, 6, 5, 2, 1],
         [7, 3, 6, 0, 4, 1], [3, 5, 4, 2, 6, 0], [3, 0, 7, 2, 4, 5],
         [5, 0, 1, 2, 6, 3]]
_ROWS_IN = [b * _V + i for b in range(_B) for i in range(_NIN)]
_ROWS_TG = [b * _V + v for b in range(_B) for v in _PERM[b]]


def _body(img, lbl, small, rows_in, rows_tg,
          img_in, img_tg, lbl_in, lbl_tg, small_in, small_tg,
          rin_v, rtg_v, sbuf_in, sbuf_tg, sem, csem):
    wid = lax.axis_index("s") * _NC + lax.axis_index("c")

    for w in range(_NWORK):
        @pl.when(wid == w)
        def _(w=w):
            for j in range(_IN_PW):
                d = w * _IN_PW + j
                s = _ROWS_IN[d]
                pltpu.async_copy(img.at[s], img_in.at[d], csem)
                pltpu.async_copy(lbl.at[s], lbl_in.at[d], csem)
            for j in range(_TG_PW):
                d = w * _TG_PW + j
                s = _ROWS_TG[d]
                pltpu.async_copy(img.at[s], img_tg.at[d], csem)
                pltpu.async_copy(lbl.at[s], lbl_tg.at[d], csem)
    for _ in range(2 * (_IN_PW + _TG_PW)):
        pltpu.make_async_copy(img.at[0], img_in.at[0], csem).wait()

    @pl.when(wid == 0)
    def _():
        pltpu.sync_copy(rows_in, rin_v)
        pltpu.async_copy(small.at[rin_v], sbuf_in, sem).wait()
        pltpu.sync_copy(sbuf_in, small_in)

    @pl.when(wid == 1)
    def _():
        pltpu.sync_copy(rows_tg, rtg_v)
        pltpu.async_copy(small.at[rtg_v], sbuf_tg, sem).wait()
        pltpu.sync_copy(sbuf_tg, small_tg)


_copy = pl.kernel(
    _body,
    out_type=(
        jax.ShapeDtypeStruct((_B * _NIN, _ROW), jnp.float32),
        jax.ShapeDtypeStruct((_B * _NTG, _ROW), jnp.float32),
        jax.ShapeDtypeStruct((_B * _NIN, _ROW), jnp.float32),
        jax.ShapeDtypeStruct((_B * _NTG, _ROW), jnp.float32),
        jax.ShapeDtypeStruct((_B * _NIN, 128), jnp.float32),
        jax.ShapeDtypeStruct((_B * _NTG, 128), jnp.float32),
    ),
    mesh=plsc.VectorSubcoreMesh(core_axis_name="c", subcore_axis_name="s"),
    scratch_types=[
        pltpu.VMEM((_B * _NIN,), jnp.int32),
        pltpu.VMEM((_B * _NTG,), jnp.int32),
        pltpu.VMEM((_B * _NIN, 128), jnp.float32),
        pltpu.VMEM((_B * _NTG, 128), jnp.float32),
        pltpu.SemaphoreType.DMA,
        pltpu.SemaphoreType.DMA,
    ],
)


def kernel(image, fxfycxcy, c2w, label):
    rows_in = jnp.asarray(_ROWS_IN, dtype=jnp.int32)
    rows_tg = jnp.asarray(_ROWS_TG, dtype=jnp.int32)
    img = image.reshape(_B * _V, _ROW)
    lbl = label.reshape(_B * _V, _ROW)
    small = jnp.pad(
        jnp.concatenate([fxfycxcy.reshape(_B * _V, 4),
                         c2w.reshape(_B * _V, 16)], axis=1),
        ((0, 0), (0, 108)))
    (img_in, img_tg, lbl_in, lbl_tg,
     small_in, small_tg) = _copy(img, lbl, small, rows_in, rows_tg)
    return (
        img_in.reshape(_B, _NIN, _C, _H, _W),
        small_in[:, :4].reshape(_B, _NIN, 4),
        small_in[:, 4:20].reshape(_B, _NIN, 4, 4),
        lbl_in.reshape(_B, _NIN, _C, _H, _W),
        img_tg.reshape(_B, _NTG, _C, _H, _W),
        small_tg[:, :4].reshape(_B, _NTG, 4),
        small_tg[:, 4:20].reshape(_B, _NTG, 4, 4),
        lbl_tg.reshape(_B, _NTG, _C, _H, _W),
    )


# speedup vs baseline: 12.0628x; 12.0628x over previous
import functools

import jax
import jax.numpy as jnp
from jax import lax
from jax.experimental import pallas as pl
from jax.experimental.pallas import tpu as pltpu
from jax.experimental.pallas import tpu_sc as plsc

_B, _V, _C, _H, _W = 16, 8, 3, 256, 256
_NIN, _NTG = 4, 6
_ROW = _C * _H * _W
_CHUNK = 3072
_CPR = _ROW // _CHUNK
_GRP = 16
_NC, _NS = 2, 16
_NWORK = _NC * _NS
_IN_CH = _B * _NIN * _CPR
_TG_CH = _B * _NTG * _CPR
_IN_PW = _IN_CH // _NWORK
_TG_PW = _TG_CH // _NWORK
_IN_G = _IN_PW // _GRP
_TG_G = _TG_PW // _GRP


def _body(img, lbl, small, tbl_in, tbl_tg, rows_in, rows_tg,
          img_in, img_tg, lbl_in, lbl_tg, small_in, small_tg,
          idx_in_v, idx_tg_v, buf0, buf1, rin_v, rtg_v, sbuf_in, sbuf_tg,
          sem, isem0, isem1, osem0, osem1):
    wid = lax.axis_index("s") * _NC + lax.axis_index("c")

    pltpu.sync_copy(tbl_in.at[wid], idx_in_v)
    pltpu.sync_copy(tbl_tg.at[wid], idx_tg_v)

    def stream(src, idx_v, n_groups, out, base_chunk):
        def gather(g, b, isem):
            pltpu.async_copy(src.at[idx_v[pl.ds(g * _GRP, _GRP)]], b, isem)

        def wait_gather(b, isem):
            pltpu.make_async_copy(src.at[pl.ds(0, _GRP)], b, isem).wait()

        def write(g, b, osem):
            pltpu.async_copy(b, out.at[pl.ds(base_chunk + g * _GRP, _GRP)],
                             osem)

        def wait_write(b, osem):
            pltpu.make_async_copy(b, out.at[pl.ds(base_chunk, _GRP)],
                                  osem).wait()

        gather(0, buf0, isem0)
        gather(1, buf1, isem1)

        def pair(k, carry):
            g = 2 * k
            wait_gather(buf0, isem0)
            write(g, buf0, osem0)
            wait_gather(buf1, isem1)
            write(g + 1, buf1, osem1)

            @pl.when(g + 2 < n_groups)
            def _():
                wait_write(buf0, osem0)
                gather(g + 2, buf0, isem0)
                wait_write(buf1, osem1)
                gather(g + 3, buf1, isem1)
            return carry

        lax.fori_loop(0, n_groups // 2, pair, 0)
        wait_write(buf0, osem0)
        wait_write(buf1, osem1)

    stream(img, idx_in_v, _IN_G, img_in, wid * _IN_PW)
    stream(img, idx_tg_v, _TG_G, img_tg, wid * _TG_PW)
    stream(lbl, idx_in_v, _IN_G, lbl_in, wid * _IN_PW)
    stream(lbl, idx_tg_v, _TG_G, lbl_tg, wid * _TG_PW)

    @pl.when(wid == 0)
    def _():
        pltpu.sync_copy(rows_in, rin_v)
        pltpu.async_copy(small.at[rin_v], sbuf_in, sem).wait()
        pltpu.sync_copy(sbuf_in, small_in)

    @pl.when(wid == 1)
    def _():
        pltpu.sync_copy(rows_tg, rtg_v)
        pltpu.async_copy(small.at[rtg_v], sbuf_tg, sem).wait()
        pltpu.sync_copy(sbuf_tg, small_tg)


_copy = pl.kernel(
    _body,
    out_type=(
        jax.ShapeDtypeStruct((_IN_CH, _CHUNK), jnp.float32),
        jax.ShapeDtypeStruct((_TG_CH, _CHUNK), jnp.float32),
        jax.ShapeDtypeStruct((_IN_CH, _CHUNK), jnp.float32),
        jax.ShapeDtypeStruct((_TG_CH, _CHUNK), jnp.float32),
        jax.ShapeDtypeStruct((_B * _NIN, 128), jnp.float32),
        jax.ShapeDtypeStruct((_B * _NTG, 128), jnp.float32),
    ),
    mesh=plsc.VectorSubcoreMesh(core_axis_name="c", subcore_axis_name="s"),
    scratch_types=[
        pltpu.VMEM((_IN_PW,), jnp.int32),
        pltpu.VMEM((_TG_PW,), jnp.int32),
        pltpu.VMEM((_GRP, _CHUNK), jnp.float32),
        pltpu.VMEM((_GRP, _CHUNK), jnp.float32),
        pltpu.VMEM((_B * _NIN,), jnp.int32),
        pltpu.VMEM((_B * _NTG,), jnp.int32),
        pltpu.VMEM((_B * _NIN, 128), jnp.float32),
        pltpu.VMEM((_B * _NTG, 128), jnp.float32),
        pltpu.SemaphoreType.DMA,
        pltpu.SemaphoreType.DMA,
        pltpu.SemaphoreType.DMA,
        pltpu.SemaphoreType.DMA,
        pltpu.SemaphoreType.DMA,
    ],
)


def _index_tables():
    u = jax.random.uniform(jax.random.key(42), (_B, _V))
    perm = jnp.argsort(u, axis=1)[:, :_NTG].astype(jnp.int32)
    base = jnp.arange(_B, dtype=jnp.int32)[:, None] * _V
    rows_in = (base + jnp.arange(_NIN, dtype=jnp.int32)[None, :]).reshape(-1)
    rows_tg = (base + perm).reshape(-1)
    carange = jnp.arange(_CPR, dtype=jnp.int32)[None, :]
    tbl_in = (rows_in[:, None] * _CPR + carange).reshape(_NWORK, _IN_PW)
    tbl_tg = (rows_tg[:, None] * _CPR + carange).reshape(_NWORK, _TG_PW)
    return rows_in, rows_tg, tbl_in, tbl_tg


def kernel(image, fxfycxcy, c2w, label):
    rows_in, rows_tg, tbl_in, tbl_tg = _index_tables()
    img = image.reshape(_B * _V * _CPR, _CHUNK)
    lbl = label.reshape(_B * _V * _CPR, _CHUNK)
    small = jnp.pad(
        jnp.concatenate([fxfycxcy.reshape(_B * _V, 4),
                         c2w.reshape(_B * _V, 16)], axis=1),
        ((0, 0), (0, 108)))
    (img_in, img_tg, lbl_in, lbl_tg,
     small_in, small_tg) = _copy(img, lbl, small,
                                 tbl_in, tbl_tg, rows_in, rows_tg)
    return (
        img_in.reshape(_B, _NIN, _C, _H, _W),
        small_in[:, :4].reshape(_B, _NIN, 4),
        small_in[:, 4:20].reshape(_B, _NIN, 4, 4),
        lbl_in.reshape(_B, _NIN, _C, _H, _W),
        img_tg.reshape(_B, _NTG, _C, _H, _W),
        small_tg[:, :4].reshape(_B, _NTG, 4),
        small_tg[:, 4:20].reshape(_B, _NTG, 4, 4),
        lbl_tg.reshape(_B, _NTG, _C, _H, _W),
    )


# speedup vs baseline: 38.8498x; 3.2206x over previous
import functools

import jax
import jax.numpy as jnp
import numpy as np
from jax import lax
from jax.experimental import pallas as pl
from jax.experimental.pallas import tpu as pltpu
from jax.experimental.pallas import tpu_sc as plsc

_B, _V, _C, _H, _W = 16, 8, 3, 256, 256
_NIN, _NTG = 4, 6
_BPR = _C * 2
_NC, _NS = 2, 16
_NWORK = _NC * _NS
_IN_BLK = _B * _NIN * _BPR
_TG_BLK = _B * _NTG * _BPR
_IN_PW = _IN_BLK // _NWORK
_TG_PW = _TG_BLK // _NWORK

_PERM = [[6, 4, 0, 3, 5, 2], [2, 6, 0, 7, 1, 5], [6, 2, 5, 4, 1, 7],
         [2, 3, 0, 7, 4, 6], [7, 6, 0, 3, 2, 5], [5, 0, 2, 3, 1, 4],
         [1, 4, 3, 7, 2, 6], [4, 5, 1, 7, 0, 2], [1, 3, 7, 5, 6, 0],
         [2, 5, 1, 6, 4, 0], [6, 4, 5, 0, 3, 2], [3, 4, 6, 5, 2, 1],
         [7, 3, 6, 0, 4, 1], [3, 5, 4, 2, 6, 0], [3, 0, 7, 2, 4, 5],
         [5, 0, 1, 2, 6, 3]]
_ROWS_IN = np.array([b * _V + i for b in range(_B) for i in range(_NIN)],
                    dtype=np.int32)
_ROWS_TG = np.array([b * _V + v for b in range(_B) for v in _PERM[b]],
                    dtype=np.int32)


def _blk_tbl(rows, per_w):
    blocks = (rows[:, None] * _BPR
              + np.arange(_BPR, dtype=np.int32)[None, :]).reshape(-1)
    return blocks.reshape(_NWORK, per_w, 1)


_TBL_IN = _blk_tbl(_ROWS_IN, _IN_PW)
_TBL_TG = _blk_tbl(_ROWS_TG, _TG_PW)


def _body(img, lbl, small, tbl_in, tbl_tg, rows_in, rows_tg,
          img_in, img_tg, lbl_in, lbl_tg, small_in, small_tg,
          idx_in_v, idx_tg_v, buf0, buf1, rin_v, rtg_v, sbuf_in, sbuf_tg,
          sem, isem0, isem1, osem0, osem1):
    wid = lax.axis_index("s") * _NC + lax.axis_index("c")

    pltpu.sync_copy(tbl_in.at[wid], idx_in_v)
    pltpu.sync_copy(tbl_tg.at[wid], idx_tg_v)

    def stream(src, idx_v, n_blocks, out, base_block):
        def gather(g, b, isem):
            pltpu.async_copy(src.at[idx_v.at[g]], b, isem)

        def wait_gather(b, isem):
            pltpu.make_async_copy(src.at[pl.ds(0, 1)], b, isem).wait()

        def write(g, b, osem):
            pltpu.async_copy(b, out.at[pl.ds(base_block + g, 1)], osem)

        def wait_write(b, osem):
            pltpu.make_async_copy(b, out.at[pl.ds(base_block, 1)],
                                  osem).wait()

        gather(0, buf0, isem0)
        gather(1, buf1, isem1)

        def pair(k, carry):
            g = 2 * k
            wait_gather(buf0, isem0)
            write(g, buf0, osem0)
            wait_gather(buf1, isem1)
            write(g + 1, buf1, osem1)

            @pl.when(g + 2 < n_blocks)
            def _():
                wait_write(buf0, osem0)
                gather(g + 2, buf0, isem0)
                wait_write(buf1, osem1)
                gather(g + 3, buf1, isem1)
            return carry

        lax.fori_loop(0, n_blocks // 2, pair, 0)
        wait_write(buf0, osem0)
        wait_write(buf1, osem1)

    stream(img, idx_in_v, _IN_PW, img_in, wid * _IN_PW)
    stream(img, idx_tg_v, _TG_PW, img_tg, wid * _TG_PW)
    stream(lbl, idx_in_v, _IN_PW, lbl_in, wid * _IN_PW)
    stream(lbl, idx_tg_v, _TG_PW, lbl_tg, wid * _TG_PW)

    @pl.when(wid == 0)
    def _():
        pltpu.sync_copy(rows_in, rin_v)
        pltpu.async_copy(small.at[rin_v], sbuf_in, sem).wait()
        pltpu.sync_copy(sbuf_in, small_in)

    @pl.when(wid == 1)
    def _():
        pltpu.sync_copy(rows_tg, rtg_v)
        pltpu.async_copy(small.at[rtg_v], sbuf_tg, sem).wait()
        pltpu.sync_copy(sbuf_tg, small_tg)


_copy = pl.kernel(
    _body,
    out_type=(
        jax.ShapeDtypeStruct((_IN_BLK, 128, 256), jnp.float32),
        jax.ShapeDtypeStruct((_TG_BLK, 128, 256), jnp.float32),
        jax.ShapeDtypeStruct((_IN_BLK, 128, 256), jnp.float32),
        jax.ShapeDtypeStruct((_TG_BLK, 128, 256), jnp.float32),
        jax.ShapeDtypeStruct((_B * _NIN, 128), jnp.float32),
        jax.ShapeDtypeStruct((_B * _NTG, 128), jnp.float32),
    ),
    mesh=plsc.VectorSubcoreMesh(core_axis_name="c", subcore_axis_name="s"),
    scratch_types=[
        pltpu.VMEM((_IN_PW, 1), jnp.int32),
        pltpu.VMEM((_TG_PW, 1), jnp.int32),
        pltpu.VMEM((1, 128, 256), jnp.float32),
        pltpu.VMEM((1, 128, 256), jnp.float32),
        pltpu.VMEM((_B * _NIN,), jnp.int32),
        pltpu.VMEM((_B * _NTG,), jnp.int32),
        pltpu.VMEM((_B * _NIN, 128), jnp.float32),
        pltpu.VMEM((_B * _NTG, 128), jnp.float32),
        pltpu.SemaphoreType.DMA,
        pltpu.SemaphoreType.DMA,
        pltpu.SemaphoreType.DMA,
        pltpu.SemaphoreType.DMA,
        pltpu.SemaphoreType.DMA,
    ],
)


def kernel(image, fxfycxcy, c2w, label):
    img = image.reshape(_B * _V * _BPR, 128, 256)
    lbl = label.reshape(_B * _V * _BPR, 128, 256)
    small = jnp.pad(
        jnp.concatenate([fxfycxcy.reshape(_B * _V, 4),
                         c2w.reshape(_B * _V, 16)], axis=1),
        ((0, 0), (0, 108)))
    (img_in, img_tg, lbl_in, lbl_tg,
     small_in, small_tg) = _copy(img, lbl, small,
                                 jnp.asarray(_TBL_IN), jnp.asarray(_TBL_TG),
                                 jnp.asarray(_ROWS_IN), jnp.asarray(_ROWS_TG))
    return (
        img_in.reshape(_B, _NIN, _C, _H, _W),
        small_in[:, :4].reshape(_B, _NIN, 4),
        small_in[:, 4:20].reshape(_B, _NIN, 4, 4),
        lbl_in.reshape(_B, _NIN, _C, _H, _W),
        img_tg.reshape(_B, _NTG, _C, _H, _W),
        small_tg[:, :4].reshape(_B, _NTG, 4),
        small_tg[:, 4:20].reshape(_B, _NTG, 4, 4),
        lbl_tg.reshape(_B, _NTG, _C, _H, _W),
    )
